# baseline (speedup 1.0000x reference)
import functools

import jax
import jax.numpy as jnp
from jax import lax
from jax.experimental import pallas as pl
from jax.experimental.pallas import tpu as pltpu
from jax.experimental.pallas import tpu_sc as plsc

_N_E = 256
_E_DIM = 1024
_BETA = 0.25
_ROWS = 8192
_R_BLK = 512

_NW = 32
_HROWS = _ROWS // 2
_ROWS_PER_W = _HROWS // _NW
_CHUNK = 32
_NCHUNK = _ROWS_PER_W // _CHUNK
_PK = 512


def _vq_body(zb_ref, embt_ref, ee_ref, oh_ref, idx_ref, mind_ref, cnt_ref):
    zb = zb_ref[...]
    s = jnp.dot(zb, embt_ref[...], preferred_element_type=jnp.float32)
    zz = jnp.sum(zb * zb, axis=1, keepdims=True)
    d = (zz + ee_ref[...]) - 2.0 * s
    mind = jnp.min(d, axis=1, keepdims=True)
    iota = lax.broadcasted_iota(jnp.int32, d.shape, 1)
    idx = jnp.min(jnp.where(d == mind, iota, _N_E), axis=1)
    oh = (iota == idx[:, None]).astype(jnp.float32)
    oh_ref[...] = oh
    idx_ref[0, 0, :] = idx
    mind_ref[0, 0, :] = mind[:, 0]
    row0 = lax.broadcasted_iota(jnp.int32, (8, _N_E), 0) == 0
    contrib = jnp.where(
        row0, jnp.broadcast_to(jnp.sum(oh, axis=0)[None, :], (8, _N_E)), 0.0
    )

    @pl.when(pl.program_id(0) == 0)
    def _init():
        cnt_ref[...] = contrib

    @pl.when(pl.program_id(0) != 0)
    def _acc():
        cnt_ref[...] += contrib


def _conv_body(emb2_ref, wt_ref, b_ref, out_ref):
    out_ref[...] = (
        jnp.dot(emb2_ref[...], wt_ref[...], preferred_element_type=jnp.float32)
        + b_ref[...]
    )


def _sc_unpack(gb, ob):
    def body(r, carry):
        for j in range(_PK // 16):
            w = gb[r, pl.ds(j * 16, 16)]
            ob[r, pl.ds(j * 16, 16)] = w << 16
            ob[r, pl.ds(_PK + j * 16, 16)] = w & jnp.int32(-65536)
        return carry

    lax.fori_loop(0, _CHUNK, body, 0)


def _sc_gather(table_hbm, idx_hbm, out_hbm, idx_v, g0, g1, o0, o1, gs0, gs1, ws0, ws1):
    wid = lax.axis_index("c") * 16 + lax.axis_index("s")
    base = wid * _ROWS_PER_W
    pltpu.sync_copy(idx_hbm.at[wid], idx_v)
    gbufs = (g0, g1)
    obufs = (o0, o1)
    gsems = (gs0, gs1)
    wsems = (ws0, ws1)
    gh, wh = {}, {}

    def start_gather(c):
        gh[c] = pltpu.async_copy(
            table_hbm.at[idx_v.at[c]], gbufs[c % 2], gsems[c % 2]
        )

    start_gather(0)
    if _NCHUNK > 1:
        start_gather(1)
    for c in range(_NCHUNK):
        gh[c].wait()
        if c - 2 in wh:
            wh[c - 2].wait()
        _sc_unpack(gbufs[c % 2], obufs[c % 2])
        wh[c] = pltpu.async_copy(
            obufs[c % 2],
            out_hbm.at[pl.ds(base + c * _CHUNK, _CHUNK)],
            wsems[c % 2],
        )
        if c + 2 < _NCHUNK:
            start_gather(c + 2)
    for c in (_NCHUNK - 2, _NCHUNK - 1):
        if c >= 0:
            wh[c].wait()


@functools.lru_cache(maxsize=1)
def _sc_gather_call():
    return pl.kernel(
        _sc_gather,
        out_type=jax.ShapeDtypeStruct((_HROWS, _E_DIM), jnp.int32),
        mesh=plsc.VectorSubcoreMesh(core_axis_name="c", subcore_axis_name="s"),
        scratch_types=[
            pltpu.VMEM((_NCHUNK, _CHUNK), jnp.int32),
            pltpu.VMEM((_CHUNK, _PK), jnp.int32),
            pltpu.VMEM((_CHUNK, _PK), jnp.int32),
            pltpu.VMEM((_CHUNK, _E_DIM), jnp.int32),
            pltpu.VMEM((_CHUNK, _E_DIM), jnp.int32),
            pltpu.SemaphoreType.DMA,
            pltpu.SemaphoreType.DMA,
            pltpu.SemaphoreType.DMA,
            pltpu.SemaphoreType.DMA,
        ],
    )


def kernel(z, emb_w, conv_w, conv_b):
    emb_t = emb_w.T
    ee = jnp.sum(emb_w**2, axis=1)[None, :]

    emb2 = emb_w.reshape(2 * _N_E, 512)
    wt = conv_w[:, :, 0, 0].T
    emb_conv = pl.pallas_call(
        _conv_body,
        out_shape=jax.ShapeDtypeStruct((2 * _N_E, 512), jnp.float32),
    )(emb2, wt, conv_b[None, :]).reshape(_N_E, _E_DIM)

    ebf = emb_conv.astype(jnp.bfloat16).astype(jnp.float32)
    eu = lax.bitcast_convert_type(ebf, jnp.uint32)
    emb_pk = lax.bitcast_convert_type(
        (eu[:, :_PK] >> 16) | (eu[:, _PK:] & jnp.uint32(0xFFFF0000)), jnp.int32
    )

    grid = _HROWS // _R_BLK
    halves = []
    for h in range(2):
        zh = lax.slice_in_dim(z, 8 * h, 8 * h + 8, axis=0)
        z_flat = jnp.transpose(zh, (0, 2, 3, 1)).reshape(_HROWS, _E_DIM)
        oh, idx3, mind3, c8 = pl.pallas_call(
            _vq_body,
            grid=(grid,),
            in_specs=[
                pl.BlockSpec((_R_BLK, _E_DIM), lambda i: (i, 0)),
                pl.BlockSpec((_E_DIM, _N_E), lambda i: (0, 0)),
                pl.BlockSpec((1, _N_E), lambda i: (0, 0)),
            ],
            out_specs=[
                pl.BlockSpec((_R_BLK, _N_E), lambda i: (i, 0)),
                pl.BlockSpec((1, 1, _R_BLK), lambda i: (i, 0, 0)),
                pl.BlockSpec((1, 1, _R_BLK), lambda i: (i, 0, 0)),
                pl.BlockSpec((8, _N_E), lambda i: (0, 0)),
            ],
            out_shape=[
                jax.ShapeDtypeStruct((_HROWS, _N_E), jnp.float32),
                jax.ShapeDtypeStruct((grid, 1, _R_BLK), jnp.int32),
                jax.ShapeDtypeStruct((grid, 1, _R_BLK), jnp.float32),
                jax.ShapeDtypeStruct((8, _N_E), jnp.float32),
            ],
        )(z_flat, emb_t, ee)
        out_i = _sc_gather_call()(emb_pk, idx3.reshape(_NW, _NCHUNK, _CHUNK))
        out_flat = lax.bitcast_convert_type(out_i, jnp.float32)
        outh = out_flat.reshape(8, 32, 32, 512).transpose(0, 3, 1, 2)
        halves.append((oh, idx3, mind3, c8, outh))

    out = jnp.concatenate([halves[0][4], halves[1][4]], axis=0)
    onehot = jnp.concatenate([halves[0][0], halves[1][0]], axis=0)
    indices = jnp.concatenate(
        [halves[0][1].reshape(_HROWS, 1), halves[1][1].reshape(_HROWS, 1)], axis=0
    )
    mind_sum = jnp.sum(halves[0][2]) + jnp.sum(halves[1][2])
    m = mind_sum / (_ROWS * _E_DIM)
    loss = m + _BETA * m
    e_mean = jnp.sum(halves[0][3] + halves[1][3], axis=0) / _ROWS
    perplexity = jnp.exp(-jnp.sum(e_mean * jnp.log(e_mean + 1e-10)))
    return (out, loss, perplexity, onehot, indices)


# speedup vs baseline: 1.2041x; 1.2041x over previous
import functools

import jax
import jax.numpy as jnp
from jax import lax
from jax.experimental import pallas as pl
from jax.experimental.pallas import tpu as pltpu
from jax.experimental.pallas import tpu_sc as plsc

_N_E = 256
_E_DIM = 1024
_BETA = 0.25
_ROWS = 8192
_R_BLK = 512

_NW = 32
_ROWS_PER_W = _ROWS // _NW
_CHUNK = 32
_NCHUNK = _ROWS_PER_W // _CHUNK
_NBUF = 3


def _vq_body(zb_ref, embt_ref, ee_ref, oh_ref, idx_ref, mind_ref, cnt_ref):
    zb = zb_ref[...]
    s = jnp.dot(zb, embt_ref[...], preferred_element_type=jnp.float32)
    zz = jnp.sum(zb * zb, axis=1, keepdims=True)
    d = (zz + ee_ref[...]) - 2.0 * s
    mind = jnp.min(d, axis=1, keepdims=True)
    iota = lax.broadcasted_iota(jnp.int32, d.shape, 1)
    idx = jnp.min(jnp.where(d == mind, iota, _N_E), axis=1)
    oh = (iota == idx[:, None]).astype(jnp.float32)
    oh_ref[...] = oh
    idx_ref[0, 0, :] = idx
    mind_ref[0, 0, :] = mind[:, 0]
    row0 = lax.broadcasted_iota(jnp.int32, (8, _N_E), 0) == 0
    contrib = jnp.where(
        row0, jnp.broadcast_to(jnp.sum(oh, axis=0)[None, :], (8, _N_E)), 0.0
    )

    @pl.when(pl.program_id(0) == 0)
    def _init():
        cnt_ref[...] = contrib

    @pl.when(pl.program_id(0) != 0)
    def _acc():
        cnt_ref[...] += contrib


def _conv_body(emb2_ref, wt_ref, b_ref, out_ref):
    out_ref[...] = (
        jnp.dot(emb2_ref[...], wt_ref[...], preferred_element_type=jnp.float32)
        + b_ref[...]
    )


def _sc_gather(table_hbm, idx_hbm, out_hbm, idx_v, b0, b1, b2, g0, g1, g2, w0, w1, w2):
    wid = lax.axis_index("c") * 16 + lax.axis_index("s")
    base = wid * _ROWS_PER_W
    pltpu.sync_copy(idx_hbm.at[wid], idx_v)
    bufs = (b0, b1, b2)
    gsems = (g0, g1, g2)
    wsems = (w0, w1, w2)
    gh, wh = {}, {}

    def start_gather(c):
        gh[c] = pltpu.async_copy(
            table_hbm.at[idx_v.at[c]], bufs[c % _NBUF], gsems[c % _NBUF]
        )

    start_gather(0)
    if _NCHUNK > 1:
        start_gather(1)
    for c in range(_NCHUNK):
        gh[c].wait()
        wh[c] = pltpu.async_copy(
            bufs[c % _NBUF],
            out_hbm.at[pl.ds(base + c * _CHUNK, _CHUNK)],
            wsems[c % _NBUF],
        )
        nxt = c + 2
        if nxt < _NCHUNK:
            if nxt - _NBUF in wh:
                wh[nxt - _NBUF].wait()
            start_gather(nxt)
    for c in range(max(0, _NCHUNK - _NBUF), _NCHUNK):
        wh[c].wait()


@functools.lru_cache(maxsize=1)
def _sc_gather_call():
    return pl.kernel(
        _sc_gather,
        out_type=jax.ShapeDtypeStruct((_ROWS, _E_DIM), jnp.float32),
        mesh=plsc.VectorSubcoreMesh(core_axis_name="c", subcore_axis_name="s"),
        scratch_types=[
            pltpu.VMEM((_NCHUNK, _CHUNK), jnp.int32),
            pltpu.VMEM((_CHUNK, _E_DIM), jnp.float32),
            pltpu.VMEM((_CHUNK, _E_DIM), jnp.float32),
            pltpu.VMEM((_CHUNK, _E_DIM), jnp.float32),
            pltpu.SemaphoreType.DMA,
            pltpu.SemaphoreType.DMA,
            pltpu.SemaphoreType.DMA,
            pltpu.SemaphoreType.DMA,
            pltpu.SemaphoreType.DMA,
            pltpu.SemaphoreType.DMA,
        ],
    )


def kernel(z, emb_w, conv_w, conv_b):
    zshape = (16, 32, 32, 512)
    z_flat = jnp.transpose(z, (0, 2, 3, 1)).reshape(_ROWS, _E_DIM)
    emb_t = emb_w.T
    ee = jnp.sum(emb_w**2, axis=1)[None, :]

    grid = _ROWS // _R_BLK
    onehot, idx3, mind3, counts8 = pl.pallas_call(
        _vq_body,
        grid=(grid,),
        in_specs=[
            pl.BlockSpec((_R_BLK, _E_DIM), lambda i: (i, 0)),
            pl.BlockSpec((_E_DIM, _N_E), lambda i: (0, 0)),
            pl.BlockSpec((1, _N_E), lambda i: (0, 0)),
        ],
        out_specs=[
            pl.BlockSpec((_R_BLK, _N_E), lambda i: (i, 0)),
            pl.BlockSpec((1, 1, _R_BLK), lambda i: (i, 0, 0)),
            pl.BlockSpec((1, 1, _R_BLK), lambda i: (i, 0, 0)),
            pl.BlockSpec((8, _N_E), lambda i: (0, 0)),
        ],
        out_shape=[
            jax.ShapeDtypeStruct((_ROWS, _N_E), jnp.float32),
            jax.ShapeDtypeStruct((grid, 1, _R_BLK), jnp.int32),
            jax.ShapeDtypeStruct((grid, 1, _R_BLK), jnp.float32),
            jax.ShapeDtypeStruct((8, _N_E), jnp.float32),
        ],
    )(z_flat, emb_t, ee)

    indices = idx3.reshape(_ROWS, 1)

    emb2 = emb_w.reshape(2 * _N_E, 512)
    wt = conv_w[:, :, 0, 0].T
    emb_conv = pl.pallas_call(
        _conv_body,
        out_shape=jax.ShapeDtypeStruct((2 * _N_E, 512), jnp.float32),
    )(emb2, wt, conv_b[None, :]).reshape(_N_E, _E_DIM)

    idx_chunks = idx3.reshape(_NW, _NCHUNK, _CHUNK)
    out_flat = _sc_gather_call()(emb_conv, idx_chunks)
    out = out_flat.reshape(zshape).transpose(0, 3, 1, 2)

    mind = mind3.reshape(_ROWS)
    m = jnp.sum(mind) / (_ROWS * _E_DIM)
    loss = m + _BETA * m
    e_mean = jnp.sum(counts8, axis=0) / _ROWS
    perplexity = jnp.exp(-jnp.sum(e_mean * jnp.log(e_mean + 1e-10)))
    return (out, loss, perplexity, onehot, indices)


# speedup vs baseline: 1.2125x; 1.0070x over previous
import functools

import jax
import jax.numpy as jnp
from jax import lax
from jax.experimental import pallas as pl
from jax.experimental.pallas import tpu as pltpu
from jax.experimental.pallas import tpu_sc as plsc

_N_E = 256
_E_DIM = 1024
_BETA = 0.25
_ROWS = 8192
_R_BLK = 1024

_NW = 32
_ROWS_PER_W = _ROWS // _NW
_CHUNK = 32
_NCHUNK = _ROWS_PER_W // _CHUNK
_NBUF = 3


def _vq_body(zb_ref, embt_ref, ee_ref, oh_ref, idx_ref, mind_ref, cnt_ref):
    zb = zb_ref[...]
    s = jnp.dot(zb, embt_ref[...], preferred_element_type=jnp.float32)
    zz = jnp.sum(zb * zb, axis=1, keepdims=True)
    d = (zz + ee_ref[...]) - 2.0 * s
    mind = jnp.min(d, axis=1, keepdims=True)
    iota = lax.broadcasted_iota(jnp.int32, d.shape, 1)
    idx = jnp.min(jnp.where(d == mind, iota, _N_E), axis=1)
    oh = (iota == idx[:, None]).astype(jnp.float32)
    oh_ref[...] = oh
    idx_ref[0, 0, :] = idx
    mind_ref[0, 0, :] = mind[:, 0]
    row0 = lax.broadcasted_iota(jnp.int32, (8, _N_E), 0) == 0
    contrib = jnp.where(
        row0, jnp.broadcast_to(jnp.sum(oh, axis=0)[None, :], (8, _N_E)), 0.0
    )

    @pl.when(pl.program_id(0) == 0)
    def _init():
        cnt_ref[...] = contrib

    @pl.when(pl.program_id(0) != 0)
    def _acc():
        cnt_ref[...] += contrib


def _conv_body(emb2_ref, wt_ref, b_ref, out_ref):
    out_ref[...] = (
        jnp.dot(emb2_ref[...], wt_ref[...], preferred_element_type=jnp.float32)
        + b_ref[...]
    )


def _sc_gather(table_hbm, idx_hbm, out_hbm, idx_v, b0, b1, b2, g0, g1, g2, w0, w1, w2):
    wid = lax.axis_index("c") * 16 + lax.axis_index("s")
    base = wid * _ROWS_PER_W
    pltpu.sync_copy(idx_hbm.at[wid], idx_v)
    bufs = (b0, b1, b2)
    gsems = (g0, g1, g2)
    wsems = (w0, w1, w2)
    gh, wh = {}, {}

    def start_gather(c):
        gh[c] = pltpu.async_copy(
            table_hbm.at[idx_v.at[c]], bufs[c % _NBUF], gsems[c % _NBUF]
        )

    start_gather(0)
    if _NCHUNK > 1:
        start_gather(1)
    for c in range(_NCHUNK):
        gh[c].wait()
        wh[c] = pltpu.async_copy(
            bufs[c % _NBUF],
            out_hbm.at[pl.ds(base + c * _CHUNK, _CHUNK)],
            wsems[c % _NBUF],
        )
        nxt = c + 2
        if nxt < _NCHUNK:
            if nxt - _NBUF in wh:
                wh[nxt - _NBUF].wait()
            start_gather(nxt)
    for c in range(max(0, _NCHUNK - _NBUF), _NCHUNK):
        wh[c].wait()


@functools.lru_cache(maxsize=1)
def _sc_gather_call():
    return pl.kernel(
        _sc_gather,
        out_type=jax.ShapeDtypeStruct((_ROWS, _E_DIM), jnp.float32),
        mesh=plsc.VectorSubcoreMesh(core_axis_name="c", subcore_axis_name="s"),
        scratch_types=[
            pltpu.VMEM((_NCHUNK, _CHUNK), jnp.int32),
            pltpu.VMEM((_CHUNK, _E_DIM), jnp.float32),
            pltpu.VMEM((_CHUNK, _E_DIM), jnp.float32),
            pltpu.VMEM((_CHUNK, _E_DIM), jnp.float32),
            pltpu.SemaphoreType.DMA,
            pltpu.SemaphoreType.DMA,
            pltpu.SemaphoreType.DMA,
            pltpu.SemaphoreType.DMA,
            pltpu.SemaphoreType.DMA,
            pltpu.SemaphoreType.DMA,
        ],
    )


def kernel(z, emb_w, conv_w, conv_b):
    zshape = (16, 32, 32, 512)
    z_flat = jnp.transpose(z, (0, 2, 3, 1)).reshape(_ROWS, _E_DIM)
    emb_t = emb_w.T
    ee = jnp.sum(emb_w**2, axis=1)[None, :]

    grid = _ROWS // _R_BLK
    onehot, idx3, mind3, counts8 = pl.pallas_call(
        _vq_body,
        grid=(grid,),
        in_specs=[
            pl.BlockSpec((_R_BLK, _E_DIM), lambda i: (i, 0)),
            pl.BlockSpec((_E_DIM, _N_E), lambda i: (0, 0)),
            pl.BlockSpec((1, _N_E), lambda i: (0, 0)),
        ],
        out_specs=[
            pl.BlockSpec((_R_BLK, _N_E), lambda i: (i, 0)),
            pl.BlockSpec((1, 1, _R_BLK), lambda i: (i, 0, 0)),
            pl.BlockSpec((1, 1, _R_BLK), lambda i: (i, 0, 0)),
            pl.BlockSpec((8, _N_E), lambda i: (0, 0)),
        ],
        out_shape=[
            jax.ShapeDtypeStruct((_ROWS, _N_E), jnp.float32),
            jax.ShapeDtypeStruct((grid, 1, _R_BLK), jnp.int32),
            jax.ShapeDtypeStruct((grid, 1, _R_BLK), jnp.float32),
            jax.ShapeDtypeStruct((8, _N_E), jnp.float32),
        ],
    )(z_flat, emb_t, ee)

    indices = idx3.reshape(_ROWS, 1)

    emb2 = emb_w.reshape(2 * _N_E, 512)
    wt = conv_w[:, :, 0, 0].T
    emb_conv = pl.pallas_call(
        _conv_body,
        out_shape=jax.ShapeDtypeStruct((2 * _N_E, 512), jnp.float32),
    )(emb2, wt, conv_b[None, :]).reshape(_N_E, _E_DIM)

    idx_chunks = idx3.reshape(_NW, _NCHUNK, _CHUNK)
    out_flat = _sc_gather_call()(emb_conv, idx_chunks)
    out = out_flat.reshape(zshape).transpose(0, 3, 1, 2)

    mind = mind3.reshape(_ROWS)
    m = jnp.sum(mind) / (_ROWS * _E_DIM)
    loss = m + _BETA * m
    e_mean = jnp.sum(counts8, axis=0) / _ROWS
    perplexity = jnp.exp(-jnp.sum(e_mean * jnp.log(e_mean + 1e-10)))
    return (out, loss, perplexity, onehot, indices)


# speedup vs baseline: 1.4985x; 1.2359x over previous
import functools

import jax
import jax.numpy as jnp
from jax import lax
from jax.experimental import pallas as pl
from jax.experimental.pallas import tpu as pltpu
from jax.experimental.pallas import tpu_sc as plsc

_N_E = 256
_E_DIM = 1024
_BETA = 0.25
_ROWS = 8192
_R_BLK = 1024

_NW = 32
_ROWS_PER_W = _ROWS // _NW
_CHUNK = 32
_NCHUNK = _ROWS_PER_W // _CHUNK
_NBUF = 3


def _vq_body(zb_ref, embt_ref, ee_ref, oh_ref, idx_ref, mind_ref, cnt_ref):
    zb = zb_ref[...]
    s = jnp.dot(zb, embt_ref[...], preferred_element_type=jnp.float32)
    zz = jnp.sum(zb * zb, axis=1, keepdims=True)
    d = (zz + ee_ref[...]) - 2.0 * s
    mind = jnp.min(d, axis=1, keepdims=True)
    iota = lax.broadcasted_iota(jnp.int32, d.shape, 1)
    idx = jnp.min(jnp.where(d == mind, iota, _N_E), axis=1)
    oh = (iota == idx[:, None]).astype(jnp.float32)
    oh_ref[...] = oh
    idx_ref[0, 0, :] = idx
    mind_ref[0, 0, :] = mind[:, 0]
    row0 = lax.broadcasted_iota(jnp.int32, (8, _N_E), 0) == 0
    contrib = jnp.where(
        row0, jnp.broadcast_to(jnp.sum(oh, axis=0)[None, :], (8, _N_E)), 0.0
    )

    @pl.when(pl.program_id(0) == 0)
    def _init():
        cnt_ref[...] = contrib

    @pl.when(pl.program_id(0) != 0)
    def _acc():
        cnt_ref[...] += contrib


def _conv_body(emb2_ref, wt_ref, b_ref, out_ref):
    out_ref[...] = (
        jnp.dot(emb2_ref[...], wt_ref[...], preferred_element_type=jnp.float32)
        + b_ref[...]
    )


def _sc_gather(table_hbm, idx_hbm, out_hbm, idx_v, b0, b1, b2, g0, g1, g2, w0, w1, w2):
    wid = lax.axis_index("c") * 16 + lax.axis_index("s")
    base = wid * _ROWS_PER_W
    pltpu.sync_copy(idx_hbm.at[wid], idx_v)
    bufs = (b0, b1, b2)
    gsems = (g0, g1, g2)
    wsems = (w0, w1, w2)
    gh, wh = {}, {}

    rep = wid % 8

    def start_gather(c):
        gh[c] = pltpu.async_copy(
            table_hbm.at[rep].at[idx_v.at[c]], bufs[c % _NBUF], gsems[c % _NBUF]
        )

    start_gather(0)
    if _NCHUNK > 1:
        start_gather(1)
    for c in range(_NCHUNK):
        gh[c].wait()
        wh[c] = pltpu.async_copy(
            bufs[c % _NBUF],
            out_hbm.at[pl.ds(base + c * _CHUNK, _CHUNK)],
            wsems[c % _NBUF],
        )
        nxt = c + 2
        if nxt < _NCHUNK:
            if nxt - _NBUF in wh:
                wh[nxt - _NBUF].wait()
            start_gather(nxt)
    for c in range(max(0, _NCHUNK - _NBUF), _NCHUNK):
        wh[c].wait()


@functools.lru_cache(maxsize=1)
def _sc_gather_call():
    return pl.kernel(
        _sc_gather,
        out_type=jax.ShapeDtypeStruct((_ROWS, _E_DIM), jnp.float32),
        mesh=plsc.VectorSubcoreMesh(core_axis_name="c", subcore_axis_name="s"),
        scratch_types=[
            pltpu.VMEM((_NCHUNK, _CHUNK), jnp.int32),
            pltpu.VMEM((_CHUNK, _E_DIM), jnp.float32),
            pltpu.VMEM((_CHUNK, _E_DIM), jnp.float32),
            pltpu.VMEM((_CHUNK, _E_DIM), jnp.float32),
            pltpu.SemaphoreType.DMA,
            pltpu.SemaphoreType.DMA,
            pltpu.SemaphoreType.DMA,
            pltpu.SemaphoreType.DMA,
            pltpu.SemaphoreType.DMA,
            pltpu.SemaphoreType.DMA,
        ],
    )


def kernel(z, emb_w, conv_w, conv_b):
    zshape = (16, 32, 32, 512)
    z_flat = jnp.transpose(z, (0, 2, 3, 1)).reshape(_ROWS, _E_DIM)
    emb_t = emb_w.T
    ee = jnp.sum(emb_w**2, axis=1)[None, :]

    grid = _ROWS // _R_BLK
    onehot, idx3, mind3, counts8 = pl.pallas_call(
        _vq_body,
        grid=(grid,),
        in_specs=[
            pl.BlockSpec((_R_BLK, _E_DIM), lambda i: (i, 0)),
            pl.BlockSpec((_E_DIM, _N_E), lambda i: (0, 0)),
            pl.BlockSpec((1, _N_E), lambda i: (0, 0)),
        ],
        out_specs=[
            pl.BlockSpec((_R_BLK, _N_E), lambda i: (i, 0)),
            pl.BlockSpec((1, 1, _R_BLK), lambda i: (i, 0, 0)),
            pl.BlockSpec((1, 1, _R_BLK), lambda i: (i, 0, 0)),
            pl.BlockSpec((8, _N_E), lambda i: (0, 0)),
        ],
        out_shape=[
            jax.ShapeDtypeStruct((_ROWS, _N_E), jnp.float32),
            jax.ShapeDtypeStruct((grid, 1, _R_BLK), jnp.int32),
            jax.ShapeDtypeStruct((grid, 1, _R_BLK), jnp.float32),
            jax.ShapeDtypeStruct((8, _N_E), jnp.float32),
        ],
    )(z_flat, emb_t, ee)

    indices = idx3.reshape(_ROWS, 1)

    emb2 = emb_w.reshape(2 * _N_E, 512)
    wt = conv_w[:, :, 0, 0].T
    emb_conv = pl.pallas_call(
        _conv_body,
        out_shape=jax.ShapeDtypeStruct((2 * _N_E, 512), jnp.float32),
    )(emb2, wt, conv_b[None, :]).reshape(_N_E, _E_DIM)

    idx_chunks = idx3.reshape(_NW, _NCHUNK, _CHUNK)
    table8 = jnp.broadcast_to(emb_conv[None], (8, _N_E, _E_DIM)) + 0.0
    out_flat = _sc_gather_call()(table8, idx_chunks)
    out = out_flat.reshape(zshape).transpose(0, 3, 1, 2)

    mind = mind3.reshape(_ROWS)
    m = jnp.sum(mind) / (_ROWS * _E_DIM)
    loss = m + _BETA * m
    e_mean = jnp.sum(counts8, axis=0) / _ROWS
    perplexity = jnp.exp(-jnp.sum(e_mean * jnp.log(e_mean + 1e-10)))
    return (out, loss, perplexity, onehot, indices)


# speedup vs baseline: 1.5166x; 1.0121x over previous
import functools

import jax
import jax.numpy as jnp
from jax import lax
from jax.experimental import pallas as pl
from jax.experimental.pallas import tpu as pltpu
from jax.experimental.pallas import tpu_sc as plsc

_N_E = 256
_E_DIM = 1024
_BETA = 0.25
_ROWS = 8192
_R_BLK = 1024

_NW = 32
_ROWS_PER_W = _ROWS // _NW
_CHUNK = 32
_NCHUNK = _ROWS_PER_W // _CHUNK
_NBUF = 3


def _vq_body(zb_ref, embt_ref, ee_ref, oh_ref, idx_ref, mind_ref, cnt_ref):
    zb = zb_ref[...]
    s = jnp.dot(zb, embt_ref[...], preferred_element_type=jnp.float32)
    zz = jnp.sum(zb * zb, axis=1, keepdims=True)
    d = (zz + ee_ref[...]) - 2.0 * s
    mind = jnp.min(d, axis=1, keepdims=True)
    iota = lax.broadcasted_iota(jnp.int32, d.shape, 1)
    idx = jnp.min(jnp.where(d == mind, iota, _N_E), axis=1)
    oh = (iota == idx[:, None]).astype(jnp.float32)
    oh_ref[...] = oh
    idx_ref[0, 0, :] = idx
    mind_ref[0, 0, :] = mind[:, 0]
    row0 = lax.broadcasted_iota(jnp.int32, (8, _N_E), 0) == 0
    contrib = jnp.where(
        row0, jnp.broadcast_to(jnp.sum(oh, axis=0)[None, :], (8, _N_E)), 0.0
    )

    @pl.when(pl.program_id(0) == 0)
    def _init():
        cnt_ref[...] = contrib

    @pl.when(pl.program_id(0) != 0)
    def _acc():
        cnt_ref[...] += contrib


def _conv_body(emb2_ref, wt_ref, b_ref, out_ref):
    out_ref[...] = (
        jnp.dot(emb2_ref[...], wt_ref[...], preferred_element_type=jnp.float32)
        + b_ref[...]
    )


def _sc_gather(table_hbm, idx_hbm, out_hbm, idx_v, b0, b1, b2, g0, g1, g2, w0, w1, w2):
    wid = lax.axis_index("c") * 16 + lax.axis_index("s")
    base = wid * _ROWS_PER_W
    pltpu.sync_copy(idx_hbm.at[wid], idx_v)
    bufs = (b0, b1, b2)
    gsems = (g0, g1, g2)
    wsems = (w0, w1, w2)
    gh, wh = {}, {}

    rep = wid % 16

    def start_gather(c):
        gh[c] = pltpu.async_copy(
            table_hbm.at[rep].at[idx_v.at[c]], bufs[c % _NBUF], gsems[c % _NBUF]
        )

    start_gather(0)
    if _NCHUNK > 1:
        start_gather(1)
    for c in range(_NCHUNK):
        gh[c].wait()
        wh[c] = pltpu.async_copy(
            bufs[c % _NBUF],
            out_hbm.at[pl.ds(base + c * _CHUNK, _CHUNK)],
            wsems[c % _NBUF],
        )
        nxt = c + 2
        if nxt < _NCHUNK:
            if nxt - _NBUF in wh:
                wh[nxt - _NBUF].wait()
            start_gather(nxt)
    for c in range(max(0, _NCHUNK - _NBUF), _NCHUNK):
        wh[c].wait()


@functools.lru_cache(maxsize=1)
def _sc_gather_call():
    return pl.kernel(
        _sc_gather,
        out_type=jax.ShapeDtypeStruct((_ROWS, _E_DIM), jnp.float32),
        mesh=plsc.VectorSubcoreMesh(core_axis_name="c", subcore_axis_name="s"),
        scratch_types=[
            pltpu.VMEM((_NCHUNK, _CHUNK), jnp.int32),
            pltpu.VMEM((_CHUNK, _E_DIM), jnp.float32),
            pltpu.VMEM((_CHUNK, _E_DIM), jnp.float32),
            pltpu.VMEM((_CHUNK, _E_DIM), jnp.float32),
            pltpu.SemaphoreType.DMA,
            pltpu.SemaphoreType.DMA,
            pltpu.SemaphoreType.DMA,
            pltpu.SemaphoreType.DMA,
            pltpu.SemaphoreType.DMA,
            pltpu.SemaphoreType.DMA,
        ],
    )


def kernel(z, emb_w, conv_w, conv_b):
    zshape = (16, 32, 32, 512)
    z_flat = jnp.transpose(z, (0, 2, 3, 1)).reshape(_ROWS, _E_DIM)
    emb_t = emb_w.T
    ee = jnp.sum(emb_w**2, axis=1)[None, :]

    grid = _ROWS // _R_BLK
    onehot, idx3, mind3, counts8 = pl.pallas_call(
        _vq_body,
        grid=(grid,),
        in_specs=[
            pl.BlockSpec((_R_BLK, _E_DIM), lambda i: (i, 0)),
            pl.BlockSpec((_E_DIM, _N_E), lambda i: (0, 0)),
            pl.BlockSpec((1, _N_E), lambda i: (0, 0)),
        ],
        out_specs=[
            pl.BlockSpec((_R_BLK, _N_E), lambda i: (i, 0)),
            pl.BlockSpec((1, 1, _R_BLK), lambda i: (i, 0, 0)),
            pl.BlockSpec((1, 1, _R_BLK), lambda i: (i, 0, 0)),
            pl.BlockSpec((8, _N_E), lambda i: (0, 0)),
        ],
        out_shape=[
            jax.ShapeDtypeStruct((_ROWS, _N_E), jnp.float32),
            jax.ShapeDtypeStruct((grid, 1, _R_BLK), jnp.int32),
            jax.ShapeDtypeStruct((grid, 1, _R_BLK), jnp.float32),
            jax.ShapeDtypeStruct((8, _N_E), jnp.float32),
        ],
    )(z_flat, emb_t, ee)

    indices = idx3.reshape(_ROWS, 1)

    emb2 = emb_w.reshape(2 * _N_E, 512)
    wt = conv_w[:, :, 0, 0].T
    emb_conv = pl.pallas_call(
        _conv_body,
        out_shape=jax.ShapeDtypeStruct((2 * _N_E, 512), jnp.float32),
    )(emb2, wt, conv_b[None, :]).reshape(_N_E, _E_DIM)

    idx_chunks = idx3.reshape(_NW, _NCHUNK, _CHUNK)
    table8 = jnp.broadcast_to(emb_conv[None], (16, _N_E, _E_DIM)) + 0.0
    out_flat = _sc_gather_call()(table8, idx_chunks)
    out = out_flat.reshape(zshape).transpose(0, 3, 1, 2)

    mind = mind3.reshape(_ROWS)
    m = jnp.sum(mind) / (_ROWS * _E_DIM)
    loss = m + _BETA * m
    e_mean = jnp.sum(counts8, axis=0) / _ROWS
    perplexity = jnp.exp(-jnp.sum(e_mean * jnp.log(e_mean + 1e-10)))
    return (out, loss, perplexity, onehot, indices)
